# Initial kernel scaffold; baseline (speedup 1.0000x reference)
import jax
import jax.numpy as jnp
from jax.experimental import pallas as pl


def kernel(x, params, edge_index):
    raise NotImplementedError("write your pallas kernel here")


# speedup vs baseline: 67.7980x; 67.7980x over previous
import functools

import jax
import jax.numpy as jnp
from jax import lax
from jax.experimental import pallas as pl
from jax.experimental.pallas import tpu as pltpu
from jax.experimental.pallas import tpu_sc as plsc

N = 50000
E = 800000
BLK = 1024
NP = 50176
G = NP // BLK

NC = 2
NS = 16
L = 16
E_PAD = 800256
EPT = E_PAD // (NC * NS)
VECS = EPT // L
NCHUNK = 4
CH = NP // NCHUNK
SLICE = CH // NS


def _sc_hist_body(row_hbm, out_hbm, idx_v, hist_v, acc_v, tmp_v, shared):
    c = lax.axis_index("c")
    s = lax.axis_index("s")
    wid = c * NS + s
    zeros = jnp.zeros((L,), jnp.float32)
    ones = jnp.ones((L,), jnp.float32)

    def zbody(j, carry):
        hist_v[pl.ds(j * L, L)] = zeros
        return carry
    lax.fori_loop(0, NP // L, zbody, 0)

    pltpu.sync_copy(row_hbm.at[pl.ds(wid * EPT, EPT)], idx_v)

    def sbody(j, carry):
        idx = idx_v[pl.ds(j * L, L)]
        plsc.addupdate_scatter(hist_v, [idx], ones)
        return carry
    lax.fori_loop(0, VECS, sbody, 0)

    for ph in range(NCHUNK):
        pltpu.sync_copy(hist_v.at[pl.ds(ph * CH, CH)],
                        shared.at[pl.ds(s * CH, CH)])
        plsc.subcore_barrier()

        pltpu.sync_copy(shared.at[pl.ds(s * SLICE, SLICE)], acc_v)

        def rbody(r, carry):
            pltpu.sync_copy(shared.at[pl.ds(r * CH + s * SLICE, SLICE)], tmp_v)

            def abody(j, carry2):
                acc_v[pl.ds(j * L, L)] = acc_v[pl.ds(j * L, L)] + tmp_v[pl.ds(j * L, L)]
                return carry2
            lax.fori_loop(0, SLICE // L, abody, 0)
            return carry
        lax.fori_loop(1, NS, rbody, 0)

        pltpu.sync_copy(
            acc_v, out_hbm.at[pl.ds(c * NP + ph * CH + s * SLICE, SLICE)])
        plsc.subcore_barrier()


@functools.cache
def _sc_hist_build():
    return functools.partial(
        pl.kernel,
        out_type=jax.ShapeDtypeStruct((NC * NP,), jnp.float32),
        mesh=plsc.VectorSubcoreMesh(
            core_axis_name="c", subcore_axis_name="s",
            num_cores=NC, num_subcores=NS),
        scratch_types=[
            pltpu.VMEM((EPT,), jnp.int32),
            pltpu.VMEM((NP,), jnp.float32),
            pltpu.VMEM((SLICE,), jnp.float32),
            pltpu.VMEM((SLICE,), jnp.float32),
            pltpu.VMEM_SHARED((NS * CH,), jnp.float32),
        ],
        compiler_params=pltpu.CompilerParams(needs_layout_passes=False),
    )(_sc_hist_body)


def _tc_body(x_ref, c0_ref, c1_ref, ew_ref, eb_ref,
             w1_ref, b1_ref, w2_ref, b2_ref, w3_ref, b3_ref,
             pw_ref, pb_ref, qy_ref, wq_ref, bq_ref,
             wk_ref, bk_ref, wv_ref, bv_ref, hp_ref, hb_ref,
             wo_ref, bo_ref, p1w_ref, p1b_ref, lng_ref, lnb_ref,
             p2w_ref, p2b_ref, out_ref, m_sc, l_sc, a_sc):
    i = pl.program_id(0)

    @pl.when(i == 0)
    def _init():
        m_sc[...] = jnp.full((1, 256), -1e30, jnp.float32)
        l_sc[...] = jnp.zeros((1, 256), jnp.float32)
        a_sc[...] = jnp.zeros((1, 256), jnp.float32)

    f32 = jnp.float32
    x = x_ref[...]
    cnt = c0_ref[...] + c1_ref[...]
    h = jnp.dot(x, ew_ref[...], preferred_element_type=f32) + eb_ref[...]
    for lyr in range(6):
        t = jnp.maximum(
            jnp.dot(h, w1_ref[lyr], preferred_element_type=f32) + b1_ref[lyr], 0.0)
        t = jnp.maximum(
            jnp.dot(t, w2_ref[lyr], preferred_element_type=f32) + b2_ref[lyr], 0.0)
        t = jnp.dot(t, w3_ref[lyr], preferred_element_type=f32) + b3_ref[lyr]
        h = h + cnt * t

    hp = jnp.dot(h, pw_ref[...], preferred_element_type=f32) + pb_ref[...]
    q = jnp.dot(qy_ref[...], wq_ref[...], preferred_element_type=f32) + bq_ref[...]
    k = jnp.dot(hp, wk_ref[...], preferred_element_type=f32) + bk_ref[...]
    v = jnp.dot(hp, wv_ref[...], preferred_element_type=f32) + bv_ref[...]

    sh = jnp.dot(jnp.dot(k * q, hp_ref[...], preferred_element_type=f32),
                 hb_ref[...], preferred_element_type=f32) * 0.125
    rows = lax.broadcasted_iota(jnp.int32, (BLK, 256), 0)
    valid = (i * BLK + rows) < N
    sh = jnp.where(valid, sh, -1e30)
    v = jnp.where(valid, v, 0.0)

    m_old = m_sc[...]
    m_new = jnp.maximum(m_old, jnp.max(sh, axis=0, keepdims=True))
    corr = jnp.exp(m_old - m_new)
    p = jnp.exp(sh - m_new)
    l_new = l_sc[...] * corr + jnp.sum(p, axis=0, keepdims=True)
    a_new = a_sc[...] * corr + jnp.sum(p * v, axis=0, keepdims=True)
    m_sc[...] = m_new
    l_sc[...] = l_new
    a_sc[...] = a_new

    @pl.when(i == G - 1)
    def _fin():
        ctx = a_new / l_new
        pooled = jnp.maximum(
            jnp.dot(ctx, wo_ref[...], preferred_element_type=f32) + bo_ref[...], 0.0)
        p1 = jnp.maximum(
            jnp.dot(pooled, p1w_ref[...], preferred_element_type=f32) + p1b_ref[...], 0.0)
        mu = jnp.mean(p1, axis=-1, keepdims=True)
        var = jnp.mean((p1 - mu) ** 2, axis=-1, keepdims=True)
        p2 = (p1 - mu) * lax.rsqrt(var + 1e-5) * lng_ref[...] + lnb_ref[...]
        out_ref[...] = jnp.dot(p2, p2w_ref[...], preferred_element_type=f32) + p2b_ref[...]


def _full(shape):
    return pl.BlockSpec(shape, lambda i: (0,) * len(shape))


def _tc_build(interpret=False):
    in_specs = [
        pl.BlockSpec((BLK, 128), lambda i: (i, 0)),
        pl.BlockSpec((BLK, 1), lambda i: (i, 0)),
        pl.BlockSpec((BLK, 1), lambda i: (i, 0)),
        _full((128, 64)), _full((1, 64)),
        _full((6, 64, 64)), _full((6, 1, 64)),
        _full((6, 64, 128)), _full((6, 1, 128)),
        _full((6, 128, 64)), _full((6, 1, 64)),
        _full((64, 256)), _full((1, 256)),
        _full((1, 256)),
        _full((256, 256)), _full((1, 256)),
        _full((256, 256)), _full((1, 256)),
        _full((256, 256)), _full((1, 256)),
        _full((256, 4)), _full((4, 256)),
        _full((256, 256)), _full((1, 256)),
        _full((256, 64)), _full((1, 64)),
        _full((1, 64)), _full((1, 64)),
        _full((64, 1024)), _full((1, 1024)),
    ]
    return pl.pallas_call(
        _tc_body,
        grid=(G,),
        in_specs=in_specs,
        out_specs=pl.BlockSpec((1, 1024), lambda i: (0, 0)),
        out_shape=jax.ShapeDtypeStruct((1, 1024), jnp.float32),
        scratch_shapes=[pltpu.VMEM((1, 256), jnp.float32)] * 3,
        compiler_params=pltpu.CompilerParams(
            dimension_semantics=("arbitrary",)),
        interpret=interpret,
    )


def kernel(x, params, edge_index):
    row = edge_index[0]
    pad_idx = N + (jnp.arange(E_PAD - E, dtype=jnp.int32) % (NP - N))
    row_pad = jnp.concatenate([row, pad_idx])
    hist = _sc_hist_build()(row_pad)
    cnt0 = hist[:NP].reshape(NP, 1)
    cnt1 = hist[NP:].reshape(NP, 1)

    x_pad = jnp.pad(x, ((0, NP - N), (0, 0)))
    p = params
    lys = p['layers']
    w1s = jnp.stack([l['w1'] for l in lys])
    b1s = jnp.stack([l['b1'].reshape(1, -1) for l in lys])
    w2s = jnp.stack([l['w2'] for l in lys])
    b2s = jnp.stack([l['b2'].reshape(1, -1) for l in lys])
    w3s = jnp.stack([l['w3'] for l in lys])
    b3s = jnp.stack([l['b3'].reshape(1, -1) for l in lys])

    heads = jnp.arange(256, dtype=jnp.int32) // 64
    head_pool = (heads[:, None] == jnp.arange(4)[None, :]).astype(jnp.float32)
    head_bcast = head_pool.T

    return _tc_build()(
        x_pad, cnt0, cnt1,
        p['embed_w'], p['embed_b'].reshape(1, -1),
        w1s, b1s, w2s, b2s, w3s, b3s,
        p['pool_w'], p['pool_b'].reshape(1, -1),
        p['query'],
        p['wq'], p['bq'].reshape(1, -1),
        p['wk'], p['bk'].reshape(1, -1),
        p['wv'], p['bv'].reshape(1, -1),
        head_pool, head_bcast,
        p['wo'], p['bo'].reshape(1, -1),
        p['pw1'], p['pb1'].reshape(1, -1),
        p['ln_g'].reshape(1, -1), p['ln_b'].reshape(1, -1),
        p['pw2'], p['pb2'].reshape(1, -1),
    )


# speedup vs baseline: 70.6247x; 1.0417x over previous
import functools

import jax
import jax.numpy as jnp
from jax import lax
from jax.experimental import pallas as pl
from jax.experimental.pallas import tpu as pltpu
from jax.experimental.pallas import tpu_sc as plsc

N = 50000
E = 800000
BLK = 1024
NP = 50176
G = NP // BLK

NC = 2
NS = 16
L = 16
E_PAD = 802816
EPT = E_PAD // (NC * NS)
VECS = EPT // L
UNR = 8
NCHUNK = 4
CH = NP // NCHUNK
SLICE = CH // NS


def _sc_hist_body(row_hbm, out_hbm, idx_v, hist_v, acc_v, tmp_v, shared, sem):
    c = lax.axis_index("c")
    s = lax.axis_index("s")
    wid = c * NS + s
    zeros = jnp.zeros((L,), jnp.float32)
    ones = jnp.ones((L,), jnp.float32)

    cp = pltpu.async_copy(row_hbm.at[pl.ds(wid * EPT, EPT)], idx_v, sem)

    def zbody(j, carry):
        for u in range(UNR):
            hist_v[pl.ds(j * (L * UNR) + u * L, L)] = zeros
        return carry
    lax.fori_loop(0, NP // (L * UNR), zbody, 0)

    cp.wait()

    def sbody(j, carry):
        for u in range(UNR):
            idx = idx_v[pl.ds(j * (L * UNR) + u * L, L)]
            plsc.addupdate_scatter(hist_v, [idx], ones)
        return carry
    lax.fori_loop(0, VECS // UNR, sbody, 0)

    for ph in range(NCHUNK):
        pltpu.sync_copy(hist_v.at[pl.ds(ph * CH, CH)],
                        shared.at[pl.ds(s * CH, CH)])
        plsc.subcore_barrier()

        pltpu.sync_copy(shared.at[pl.ds(s * SLICE, SLICE)], acc_v)

        def rbody(r, carry):
            pltpu.sync_copy(shared.at[pl.ds(r * CH + s * SLICE, SLICE)], tmp_v)

            def abody(j, carry2):
                for u in range(7):
                    o = j * (L * 7) + u * L
                    acc_v[pl.ds(o, L)] = acc_v[pl.ds(o, L)] + tmp_v[pl.ds(o, L)]
                return carry2
            lax.fori_loop(0, SLICE // (L * 7), abody, 0)
            return carry
        lax.fori_loop(1, NS, rbody, 0)

        pltpu.sync_copy(
            acc_v, out_hbm.at[pl.ds(c * NP + ph * CH + s * SLICE, SLICE)])
        plsc.subcore_barrier()


@functools.cache
def _sc_hist_build():
    return functools.partial(
        pl.kernel,
        out_type=jax.ShapeDtypeStruct((NC * NP,), jnp.float32),
        mesh=plsc.VectorSubcoreMesh(
            core_axis_name="c", subcore_axis_name="s",
            num_cores=NC, num_subcores=NS),
        scratch_types=[
            pltpu.VMEM((EPT,), jnp.int32),
            pltpu.VMEM((NP,), jnp.float32),
            pltpu.VMEM((SLICE,), jnp.float32),
            pltpu.VMEM((SLICE,), jnp.float32),
            pltpu.VMEM_SHARED((NS * CH,), jnp.float32),
            pltpu.SemaphoreType.DMA,
        ],
        compiler_params=pltpu.CompilerParams(needs_layout_passes=False),
    )(_sc_hist_body)


def _tc_body(x_ref, c0_ref, c1_ref, ew_ref, eb_ref,
             w1_ref, b1_ref, w2_ref, b2_ref, w3_ref, b3_ref,
             pw_ref, pb_ref, qy_ref, wq_ref, bq_ref,
             wk_ref, bk_ref, wv_ref, bv_ref, hp_ref, hb_ref,
             wo_ref, bo_ref, p1w_ref, p1b_ref, lng_ref, lnb_ref,
             p2w_ref, p2b_ref, out_ref, m_sc, l_sc, a_sc,
             ws_sc, bs_sc, wv_sc, bv_sc):
    i = pl.program_id(0)
    f32 = jnp.float32

    @pl.when(i == 0)
    def _init():
        m_sc[...] = jnp.full((1, 256), -1e30, jnp.float32)
        l_sc[...] = jnp.zeros((1, 256), jnp.float32)
        a_sc[...] = jnp.zeros((1, 256), jnp.float32)
        q = jnp.dot(qy_ref[...], wq_ref[...], preferred_element_type=f32, precision=jax.lax.Precision.HIGHEST) + bq_ref[...]
        rr = lax.broadcasted_iota(jnp.int32, (256, 256), 0)
        cc = lax.broadcasted_iota(jnp.int32, (256, 256), 1)
        diag_q = jnp.where(rr == cc, jnp.ones((256, 1), f32) * q, 0.0)
        qhp = jnp.dot(diag_q, hp_ref[...], preferred_element_type=f32, precision=jax.lax.Precision.HIGHEST)
        pwk = jnp.dot(pw_ref[...], wk_ref[...], preferred_element_type=f32, precision=jax.lax.Precision.HIGHEST)
        ws_sc[...] = jnp.dot(pwk, qhp, preferred_element_type=f32, precision=jax.lax.Precision.HIGHEST)
        kb = jnp.dot(pb_ref[...], wk_ref[...], preferred_element_type=f32, precision=jax.lax.Precision.HIGHEST) + bk_ref[...]
        bs_sc[...] = jnp.dot(kb, qhp, preferred_element_type=f32, precision=jax.lax.Precision.HIGHEST)
        wv_sc[...] = jnp.dot(pw_ref[...], wv_ref[...], preferred_element_type=f32, precision=jax.lax.Precision.HIGHEST)
        bv_sc[...] = jnp.dot(pb_ref[...], wv_ref[...], preferred_element_type=f32, precision=jax.lax.Precision.HIGHEST) + bv_ref[...]

    x = x_ref[...]
    cnt = c0_ref[...] + c1_ref[...]
    h = jnp.dot(x, ew_ref[...], preferred_element_type=f32) + eb_ref[...]
    for lyr in range(6):
        t = jnp.maximum(
            jnp.dot(h, w1_ref[lyr], preferred_element_type=f32) + b1_ref[lyr], 0.0)
        t = jnp.maximum(
            jnp.dot(t, w2_ref[lyr], preferred_element_type=f32) + b2_ref[lyr], 0.0)
        t = jnp.dot(t, w3_ref[lyr], preferred_element_type=f32) + b3_ref[lyr]
        h = h + cnt * t

    v = jnp.dot(h, wv_sc[...], preferred_element_type=f32) + bv_sc[...]
    s4 = (jnp.dot(h, ws_sc[...], preferred_element_type=f32, precision=jax.lax.Precision.HIGHEST) + bs_sc[...]) * 0.125
    sh = jnp.dot(s4, hb_ref[...], preferred_element_type=f32)
    rows = lax.broadcasted_iota(jnp.int32, (BLK, 256), 0)
    valid = (i * BLK + rows) < N
    sh = jnp.where(valid, sh, -1e30)
    v = jnp.where(valid, v, 0.0)

    m_old = m_sc[...]
    m_new = jnp.maximum(m_old, jnp.max(sh, axis=0, keepdims=True))
    corr = jnp.exp(m_old - m_new)
    p = jnp.exp(sh - m_new)
    l_new = l_sc[...] * corr + jnp.sum(p, axis=0, keepdims=True)
    a_new = a_sc[...] * corr + jnp.sum(p * v, axis=0, keepdims=True)
    m_sc[...] = m_new
    l_sc[...] = l_new
    a_sc[...] = a_new

    @pl.when(i == G - 1)
    def _fin():
        ctx = a_new / l_new
        pooled = jnp.maximum(
            jnp.dot(ctx, wo_ref[...], preferred_element_type=f32) + bo_ref[...], 0.0)
        p1 = jnp.maximum(
            jnp.dot(pooled, p1w_ref[...], preferred_element_type=f32) + p1b_ref[...], 0.0)
        mu = jnp.mean(p1, axis=-1, keepdims=True)
        var = jnp.mean((p1 - mu) ** 2, axis=-1, keepdims=True)
        p2 = (p1 - mu) * lax.rsqrt(var + 1e-5) * lng_ref[...] + lnb_ref[...]
        out_ref[...] = jnp.dot(p2, p2w_ref[...], preferred_element_type=f32) + p2b_ref[...]


def _full(shape):
    return pl.BlockSpec(shape, lambda i: (0,) * len(shape))


def _tc_build(interpret=False):
    in_specs = [
        pl.BlockSpec((BLK, 128), lambda i: (i, 0)),
        pl.BlockSpec((BLK, 1), lambda i: (i, 0)),
        pl.BlockSpec((BLK, 1), lambda i: (i, 0)),
        _full((128, 64)), _full((1, 64)),
        _full((6, 64, 64)), _full((6, 1, 64)),
        _full((6, 64, 128)), _full((6, 1, 128)),
        _full((6, 128, 64)), _full((6, 1, 64)),
        _full((64, 256)), _full((1, 256)),
        _full((1, 256)),
        _full((256, 256)), _full((1, 256)),
        _full((256, 256)), _full((1, 256)),
        _full((256, 256)), _full((1, 256)),
        _full((256, 4)), _full((4, 256)),
        _full((256, 256)), _full((1, 256)),
        _full((256, 64)), _full((1, 64)),
        _full((1, 64)), _full((1, 64)),
        _full((64, 1024)), _full((1, 1024)),
    ]
    return pl.pallas_call(
        _tc_body,
        grid=(G,),
        in_specs=in_specs,
        out_specs=pl.BlockSpec((1, 1024), lambda i: (0, 0)),
        out_shape=jax.ShapeDtypeStruct((1, 1024), jnp.float32),
        scratch_shapes=[pltpu.VMEM((1, 256), jnp.float32)] * 3 + [
            pltpu.VMEM((64, 4), jnp.float32),
            pltpu.VMEM((1, 4), jnp.float32),
            pltpu.VMEM((64, 256), jnp.float32),
            pltpu.VMEM((1, 256), jnp.float32),
        ],
        compiler_params=pltpu.CompilerParams(
            dimension_semantics=("arbitrary",)),
        interpret=interpret,
    )


def kernel(x, params, edge_index):
    row = edge_index[0]
    pad_idx = N + (jnp.arange(E_PAD - E, dtype=jnp.int32) % (NP - N))
    row_pad = jnp.concatenate([row, pad_idx])
    hist = _sc_hist_build()(row_pad)
    cnt0 = hist[:NP].reshape(NP, 1)
    cnt1 = hist[NP:].reshape(NP, 1)

    x_pad = jnp.pad(x, ((0, NP - N), (0, 0)))
    p = params
    lys = p['layers']
    w1s = jnp.stack([l['w1'] for l in lys])
    b1s = jnp.stack([l['b1'].reshape(1, -1) for l in lys])
    w2s = jnp.stack([l['w2'] for l in lys])
    b2s = jnp.stack([l['b2'].reshape(1, -1) for l in lys])
    w3s = jnp.stack([l['w3'] for l in lys])
    b3s = jnp.stack([l['b3'].reshape(1, -1) for l in lys])

    heads = jnp.arange(256, dtype=jnp.int32) // 64
    head_pool = (heads[:, None] == jnp.arange(4)[None, :]).astype(jnp.float32)
    head_bcast = head_pool.T

    return _tc_build()(
        x_pad, cnt0, cnt1,
        p['embed_w'], p['embed_b'].reshape(1, -1),
        w1s, b1s, w2s, b2s, w3s, b3s,
        p['pool_w'], p['pool_b'].reshape(1, -1),
        p['query'],
        p['wq'], p['bq'].reshape(1, -1),
        p['wk'], p['bk'].reshape(1, -1),
        p['wv'], p['bv'].reshape(1, -1),
        head_pool, head_bcast,
        p['wo'], p['bo'].reshape(1, -1),
        p['pw1'], p['pb1'].reshape(1, -1),
        p['ln_g'].reshape(1, -1), p['ln_b'].reshape(1, -1),
        p['pw2'], p['pb2'].reshape(1, -1),
    )


# speedup vs baseline: 71.2350x; 1.0086x over previous
import functools

import jax
import jax.numpy as jnp
from jax import lax
from jax.experimental import pallas as pl
from jax.experimental.pallas import tpu as pltpu
from jax.experimental.pallas import tpu_sc as plsc

N = 50000
E = 800000
BLK = 1024
NP = 50176
G = NP // BLK

NC = 2
NS = 16
L = 16
E_PAD = 802816
EPT = E_PAD // (NC * NS)
VECS = EPT // L
UNR = 8
NCHUNK = 4
CH = NP // NCHUNK
SLICE = CH // NS


def _sc_hist_body(row_hbm, out_hbm, idx_v, hist_v, acc_v, tmp_v, shared, sem):
    c = lax.axis_index("c")
    s = lax.axis_index("s")
    wid = c * NS + s
    zeros = jnp.zeros((L,), jnp.float32)
    ones = jnp.ones((L,), jnp.float32)

    cp = pltpu.async_copy(row_hbm.at[pl.ds(wid * EPT, EPT)], idx_v, sem)

    def zbody(j, carry):
        for u in range(UNR):
            hist_v[pl.ds(j * (L * UNR) + u * L, L)] = zeros
        return carry
    lax.fori_loop(0, NP // (L * UNR), zbody, 0)

    cp.wait()

    def sbody(j, carry):
        for u in range(UNR):
            idx = idx_v[pl.ds(j * (L * UNR) + u * L, L)]
            plsc.addupdate_scatter(hist_v, [idx], ones)
        return carry
    lax.fori_loop(0, VECS // UNR, sbody, 0)

    for ph in range(NCHUNK):
        pltpu.sync_copy(hist_v.at[pl.ds(ph * CH, CH)],
                        shared.at[pl.ds(s * CH, CH)])
        plsc.subcore_barrier()

        pltpu.sync_copy(shared.at[pl.ds(s * SLICE, SLICE)], acc_v)

        def rbody(r, carry):
            pltpu.sync_copy(shared.at[pl.ds(r * CH + s * SLICE, SLICE)], tmp_v)

            def abody(j, carry2):
                for u in range(7):
                    o = j * (L * 7) + u * L
                    acc_v[pl.ds(o, L)] = acc_v[pl.ds(o, L)] + tmp_v[pl.ds(o, L)]
                return carry2
            lax.fori_loop(0, SLICE // (L * 7), abody, 0)
            return carry
        lax.fori_loop(1, NS, rbody, 0)

        pltpu.sync_copy(
            acc_v, out_hbm.at[pl.ds(c * NP + ph * CH + s * SLICE, SLICE)])
        plsc.subcore_barrier()


@functools.cache
def _sc_hist_build():
    return functools.partial(
        pl.kernel,
        out_type=jax.ShapeDtypeStruct((NC * NP,), jnp.float32),
        mesh=plsc.VectorSubcoreMesh(
            core_axis_name="c", subcore_axis_name="s",
            num_cores=NC, num_subcores=NS),
        scratch_types=[
            pltpu.VMEM((EPT,), jnp.int32),
            pltpu.VMEM((NP,), jnp.float32),
            pltpu.VMEM((SLICE,), jnp.float32),
            pltpu.VMEM((SLICE,), jnp.float32),
            pltpu.VMEM_SHARED((NS * CH,), jnp.float32),
            pltpu.SemaphoreType.DMA,
        ],
        compiler_params=pltpu.CompilerParams(needs_layout_passes=False),
    )(_sc_hist_body)


def _tc_body(x_ref, c0_ref, c1_ref, msk_ref, ew_ref, eb_ref,
             w1_ref, b1_ref, w2_ref, b2_ref, w3_ref, b3_ref,
             pw_ref, pb_ref, qy_ref, wq_ref, bq_ref,
             wk_ref, bk_ref, wv_ref, bv_ref, hp_ref, hb_ref,
             wo_ref, bo_ref, p1w_ref, p1b_ref, lng_ref, lnb_ref,
             p2w_ref, p2b_ref, out_ref, m_sc, l_sc, a_sc,
             ws_sc, bs_sc, wv_sc, bv_sc):
    i = pl.program_id(0)
    f32 = jnp.float32

    @pl.when(i == 0)
    def _init():
        m_sc[...] = jnp.full((1, 4), -1e30, jnp.float32)
        l_sc[...] = jnp.zeros((1, 256), jnp.float32)
        a_sc[...] = jnp.zeros((1, 256), jnp.float32)
        q = jnp.dot(qy_ref[...], wq_ref[...], preferred_element_type=f32, precision=jax.lax.Precision.HIGHEST) + bq_ref[...]
        rr = lax.broadcasted_iota(jnp.int32, (256, 256), 0)
        cc = lax.broadcasted_iota(jnp.int32, (256, 256), 1)
        diag_q = jnp.where(rr == cc, jnp.ones((256, 1), f32) * q, 0.0)
        qhp = jnp.dot(diag_q, hp_ref[...], preferred_element_type=f32, precision=jax.lax.Precision.HIGHEST)
        pwk = jnp.dot(pw_ref[...], wk_ref[...], preferred_element_type=f32, precision=jax.lax.Precision.HIGHEST)
        ws_sc[...] = jnp.dot(pwk, qhp, preferred_element_type=f32, precision=jax.lax.Precision.HIGHEST) * 0.125
        kb = jnp.dot(pb_ref[...], wk_ref[...], preferred_element_type=f32, precision=jax.lax.Precision.HIGHEST) + bk_ref[...]
        bs_sc[...] = jnp.dot(kb, qhp, preferred_element_type=f32, precision=jax.lax.Precision.HIGHEST) * 0.125
        wv_sc[...] = jnp.dot(pw_ref[...], wv_ref[...], preferred_element_type=f32, precision=jax.lax.Precision.HIGHEST)
        bv_sc[...] = jnp.dot(pb_ref[...], wv_ref[...], preferred_element_type=f32, precision=jax.lax.Precision.HIGHEST) + bv_ref[...]

    x = x_ref[...]
    cnt = c0_ref[...] + c1_ref[...]
    h = jnp.dot(x, ew_ref[...], preferred_element_type=f32) + eb_ref[...]
    for lyr in range(6):
        t = jnp.maximum(
            jnp.dot(h, w1_ref[lyr], preferred_element_type=f32) + b1_ref[lyr], 0.0)
        t = jnp.maximum(
            jnp.dot(t, w2_ref[lyr], preferred_element_type=f32) + b2_ref[lyr], 0.0)
        t = jnp.dot(t, w3_ref[lyr], preferred_element_type=f32) + b3_ref[lyr]
        h = h + cnt * t

    v = jnp.dot(h, wv_sc[...], preferred_element_type=f32) + bv_sc[...]
    s4 = (jnp.dot(h, ws_sc[...], preferred_element_type=f32,
                  precision=jax.lax.Precision.HIGHEST)
          + bs_sc[...] + msk_ref[...])

    m_old = m_sc[...]
    m_new = jnp.maximum(m_old, jnp.max(s4, axis=0, keepdims=True))
    corr256 = jnp.dot(jnp.exp(m_old - m_new), hb_ref[...],
                      preferred_element_type=f32)
    p4 = jnp.exp(s4 - m_new)
    pw = jnp.dot(p4, hb_ref[...], preferred_element_type=f32)
    ones_row = jnp.ones((1, BLK), f32)
    l_new = l_sc[...] * corr256 + jnp.dot(ones_row, pw, preferred_element_type=f32)
    a_new = a_sc[...] * corr256 + jnp.dot(ones_row, pw * v, preferred_element_type=f32)
    m_sc[...] = m_new
    l_sc[...] = l_new
    a_sc[...] = a_new

    @pl.when(i == G - 1)
    def _fin():
        ctx = a_new / l_new
        pooled = jnp.maximum(
            jnp.dot(ctx, wo_ref[...], preferred_element_type=f32) + bo_ref[...], 0.0)
        p1 = jnp.maximum(
            jnp.dot(pooled, p1w_ref[...], preferred_element_type=f32) + p1b_ref[...], 0.0)
        mu = jnp.mean(p1, axis=-1, keepdims=True)
        var = jnp.mean((p1 - mu) ** 2, axis=-1, keepdims=True)
        p2 = (p1 - mu) * lax.rsqrt(var + 1e-5) * lng_ref[...] + lnb_ref[...]
        out_ref[...] = jnp.dot(p2, p2w_ref[...], preferred_element_type=f32) + p2b_ref[...]


def _full(shape):
    return pl.BlockSpec(shape, lambda i: (0,) * len(shape))


def _tc_build(interpret=False):
    in_specs = [
        pl.BlockSpec((BLK, 128), lambda i: (i, 0)),
        pl.BlockSpec((BLK, 1), lambda i: (i, 0)),
        pl.BlockSpec((BLK, 1), lambda i: (i, 0)),
        pl.BlockSpec((BLK, 1), lambda i: (i, 0)),
        _full((128, 64)), _full((1, 64)),
        _full((6, 64, 64)), _full((6, 1, 64)),
        _full((6, 64, 128)), _full((6, 1, 128)),
        _full((6, 128, 64)), _full((6, 1, 64)),
        _full((64, 256)), _full((1, 256)),
        _full((1, 256)),
        _full((256, 256)), _full((1, 256)),
        _full((256, 256)), _full((1, 256)),
        _full((256, 256)), _full((1, 256)),
        _full((256, 4)), _full((4, 256)),
        _full((256, 256)), _full((1, 256)),
        _full((256, 64)), _full((1, 64)),
        _full((1, 64)), _full((1, 64)),
        _full((64, 1024)), _full((1, 1024)),
    ]
    return pl.pallas_call(
        _tc_body,
        grid=(G,),
        in_specs=in_specs,
        out_specs=pl.BlockSpec((1, 1024), lambda i: (0, 0)),
        out_shape=jax.ShapeDtypeStruct((1, 1024), jnp.float32),
        scratch_shapes=[
            pltpu.VMEM((1, 4), jnp.float32),
            pltpu.VMEM((1, 256), jnp.float32),
            pltpu.VMEM((1, 256), jnp.float32),
        ] + [
            pltpu.VMEM((64, 4), jnp.float32),
            pltpu.VMEM((1, 4), jnp.float32),
            pltpu.VMEM((64, 256), jnp.float32),
            pltpu.VMEM((1, 256), jnp.float32),
        ],
        compiler_params=pltpu.CompilerParams(
            dimension_semantics=("arbitrary",)),
        interpret=interpret,
    )


def kernel(x, params, edge_index):
    row = edge_index[0]
    pad_idx = N + (jnp.arange(E_PAD - E, dtype=jnp.int32) % (NP - N))
    row_pad = jnp.concatenate([row, pad_idx])
    hist = _sc_hist_build()(row_pad)
    cnt0 = hist[:NP].reshape(NP, 1)
    cnt1 = hist[NP:].reshape(NP, 1)

    x_pad = jnp.pad(x, ((0, NP - N), (0, 0)))
    p = params
    lys = p['layers']
    w1s = jnp.stack([l['w1'] for l in lys])
    b1s = jnp.stack([l['b1'].reshape(1, -1) for l in lys])
    w2s = jnp.stack([l['w2'] for l in lys])
    b2s = jnp.stack([l['b2'].reshape(1, -1) for l in lys])
    w3s = jnp.stack([l['w3'] for l in lys])
    b3s = jnp.stack([l['b3'].reshape(1, -1) for l in lys])

    heads = jnp.arange(256, dtype=jnp.int32) // 64
    head_pool = (heads[:, None] == jnp.arange(4)[None, :]).astype(jnp.float32)
    head_bcast = head_pool.T
    msk = jnp.where(jnp.arange(NP) < N, 0.0, -1e30).astype(jnp.float32).reshape(NP, 1)

    return _tc_build()(
        x_pad, cnt0, cnt1, msk,
        p['embed_w'], p['embed_b'].reshape(1, -1),
        w1s, b1s, w2s, b2s, w3s, b3s,
        p['pool_w'], p['pool_b'].reshape(1, -1),
        p['query'],
        p['wq'], p['bq'].reshape(1, -1),
        p['wk'], p['bk'].reshape(1, -1),
        p['wv'], p['bv'].reshape(1, -1),
        head_pool, head_bcast,
        p['wo'], p['bo'].reshape(1, -1),
        p['pw1'], p['pb1'].reshape(1, -1),
        p['ln_g'].reshape(1, -1), p['ln_b'].reshape(1, -1),
        p['pw2'], p['pb2'].reshape(1, -1),
    )


# speedup vs baseline: 87.2693x; 1.2251x over previous
import functools

import jax
import jax.numpy as jnp
from jax import lax
from jax.experimental import pallas as pl
from jax.experimental.pallas import tpu as pltpu
from jax.experimental.pallas import tpu_sc as plsc

N = 50000
E = 800000
BLK = 2048
NP = 50176
G = NP // BLK

NC = 2
NS = 16
L = 16
E_PAD = 802816
EPT = E_PAD // (NC * NS)
VECS = EPT // L
UNR = 8
NCHUNK = 4
CH = NP // NCHUNK
SLICE = CH // NS


def _sc_hist_body(row_hbm, out_hbm, idx_v, hist_v, acc_v, tmp_v, shared, sem):
    c = lax.axis_index("c")
    s = lax.axis_index("s")
    wid = c * NS + s
    zeros = jnp.zeros((L,), jnp.float32)
    ones = jnp.ones((L,), jnp.float32)

    cp = pltpu.async_copy(row_hbm.at[pl.ds(wid * EPT, EPT)], idx_v, sem)

    def zbody(j, carry):
        for u in range(UNR):
            hist_v[pl.ds(j * (L * UNR) + u * L, L)] = zeros
        return carry
    lax.fori_loop(0, NP // (L * UNR), zbody, 0)

    cp.wait()

    def sbody(j, carry):
        for u in range(UNR):
            idx = idx_v[pl.ds(j * (L * UNR) + u * L, L)]
            plsc.addupdate_scatter(hist_v, [idx], ones)
        return carry
    lax.fori_loop(0, VECS // UNR, sbody, 0)

    for ph in range(NCHUNK):
        pltpu.sync_copy(hist_v.at[pl.ds(ph * CH, CH)],
                        shared.at[pl.ds(s * CH, CH)])
        plsc.subcore_barrier()

        pltpu.sync_copy(shared.at[pl.ds(s * SLICE, SLICE)], acc_v)

        def rbody(r, carry):
            pltpu.sync_copy(shared.at[pl.ds(r * CH + s * SLICE, SLICE)], tmp_v)

            def abody(j, carry2):
                for u in range(7):
                    o = j * (L * 7) + u * L
                    acc_v[pl.ds(o, L)] = acc_v[pl.ds(o, L)] + tmp_v[pl.ds(o, L)]
                return carry2
            lax.fori_loop(0, SLICE // (L * 7), abody, 0)
            return carry
        lax.fori_loop(1, NS, rbody, 0)

        pltpu.sync_copy(
            acc_v, out_hbm.at[pl.ds(c * NP + ph * CH + s * SLICE, SLICE)])
        plsc.subcore_barrier()


@functools.cache
def _sc_hist_build():
    return functools.partial(
        pl.kernel,
        out_type=jax.ShapeDtypeStruct((NC * NP,), jnp.float32),
        mesh=plsc.VectorSubcoreMesh(
            core_axis_name="c", subcore_axis_name="s",
            num_cores=NC, num_subcores=NS),
        scratch_types=[
            pltpu.VMEM((EPT,), jnp.int32),
            pltpu.VMEM((NP,), jnp.float32),
            pltpu.VMEM((SLICE,), jnp.float32),
            pltpu.VMEM((SLICE,), jnp.float32),
            pltpu.VMEM_SHARED((NS * CH,), jnp.float32),
            pltpu.SemaphoreType.DMA,
        ],
        compiler_params=pltpu.CompilerParams(needs_layout_passes=False),
    )(_sc_hist_body)


def _tc_body(x_ref, c0_ref, c1_ref, msk_ref, ew_ref, eb_ref,
             w1_ref, b1_ref, w2_ref, b2_ref, w3_ref, b3_ref,
             pw_ref, pb_ref, qy_ref, wq_ref, bq_ref,
             wk_ref, bk_ref, wv_ref, bv_ref, hp_ref, hb_ref,
             wo_ref, bo_ref, p1w_ref, p1b_ref, lng_ref, lnb_ref,
             p2w_ref, p2b_ref, out_ref, m_sc, l_sc, a_sc,
             ws_sc, bs_sc, wv_sc, bv_sc):
    i = pl.program_id(0)
    f32 = jnp.float32

    @pl.when(i == 0)
    def _init():
        m_sc[...] = jnp.full((1, 4), -1e30, jnp.float32)
        l_sc[...] = jnp.zeros((1, 256), jnp.float32)
        a_sc[...] = jnp.zeros((1, 256), jnp.float32)
        q = jnp.dot(qy_ref[...], wq_ref[...], preferred_element_type=f32, precision=jax.lax.Precision.HIGHEST) + bq_ref[...]
        rr = lax.broadcasted_iota(jnp.int32, (256, 256), 0)
        cc = lax.broadcasted_iota(jnp.int32, (256, 256), 1)
        diag_q = jnp.where(rr == cc, jnp.ones((256, 1), f32) * q, 0.0)
        qhp = jnp.dot(diag_q, hp_ref[...], preferred_element_type=f32, precision=jax.lax.Precision.HIGHEST)
        pwk = jnp.dot(pw_ref[...], wk_ref[...], preferred_element_type=f32, precision=jax.lax.Precision.HIGHEST)
        ws_sc[...] = jnp.dot(pwk, qhp, preferred_element_type=f32, precision=jax.lax.Precision.HIGHEST) * 0.125
        kb = jnp.dot(pb_ref[...], wk_ref[...], preferred_element_type=f32, precision=jax.lax.Precision.HIGHEST) + bk_ref[...]
        bs_sc[...] = jnp.dot(kb, qhp, preferred_element_type=f32, precision=jax.lax.Precision.HIGHEST) * 0.125
        wv_sc[...] = jnp.dot(pw_ref[...], wv_ref[...], preferred_element_type=f32, precision=jax.lax.Precision.HIGHEST)
        bv_sc[...] = jnp.dot(pb_ref[...], wv_ref[...], preferred_element_type=f32, precision=jax.lax.Precision.HIGHEST) + bv_ref[...]

    x = x_ref[...]
    cnt = c0_ref[...] + c1_ref[...]
    h = jnp.dot(x, ew_ref[...], preferred_element_type=f32) + eb_ref[...]
    for lyr in range(6):
        t = jnp.maximum(
            jnp.dot(h, w1_ref[lyr], preferred_element_type=f32) + b1_ref[lyr], 0.0)
        t = jnp.maximum(
            jnp.dot(t, w2_ref[lyr], preferred_element_type=f32) + b2_ref[lyr], 0.0)
        t = jnp.dot(t, w3_ref[lyr], preferred_element_type=f32) + b3_ref[lyr]
        h = h + cnt * t

    v = jnp.dot(h, wv_sc[...], preferred_element_type=f32) + bv_sc[...]
    s4 = (jnp.dot(h, ws_sc[...], preferred_element_type=f32,
                  precision=jax.lax.Precision.HIGHEST)
          + bs_sc[...] + msk_ref[...])

    m_old = m_sc[...]
    m_new = jnp.maximum(m_old, jnp.max(s4, axis=0, keepdims=True))
    corr256 = jnp.dot(jnp.exp(m_old - m_new), hb_ref[...],
                      preferred_element_type=f32)
    p4 = jnp.exp(s4 - m_new)
    pw = jnp.dot(p4, hb_ref[...], preferred_element_type=f32)
    ones_row = jnp.ones((1, BLK), f32)
    l_new = l_sc[...] * corr256 + jnp.dot(ones_row, pw, preferred_element_type=f32)
    a_new = a_sc[...] * corr256 + jnp.dot(ones_row, pw * v, preferred_element_type=f32)
    m_sc[...] = m_new
    l_sc[...] = l_new
    a_sc[...] = a_new

    @pl.when(i == G - 1)
    def _fin():
        ctx = a_new / l_new
        pooled = jnp.maximum(
            jnp.dot(ctx, wo_ref[...], preferred_element_type=f32) + bo_ref[...], 0.0)
        p1 = jnp.maximum(
            jnp.dot(pooled, p1w_ref[...], preferred_element_type=f32) + p1b_ref[...], 0.0)
        mu = jnp.mean(p1, axis=-1, keepdims=True)
        var = jnp.mean((p1 - mu) ** 2, axis=-1, keepdims=True)
        p2 = (p1 - mu) * lax.rsqrt(var + 1e-5) * lng_ref[...] + lnb_ref[...]
        out_ref[...] = jnp.dot(p2, p2w_ref[...], preferred_element_type=f32) + p2b_ref[...]


def _full(shape):
    return pl.BlockSpec(shape, lambda i: (0,) * len(shape))


def _tc_build(interpret=False):
    in_specs = [
        pl.BlockSpec((BLK, 128), lambda i: (i, 0)),
        pl.BlockSpec((BLK, 1), lambda i: (i, 0)),
        pl.BlockSpec((BLK, 1), lambda i: (i, 0)),
        pl.BlockSpec((BLK, 1), lambda i: (i, 0)),
        _full((128, 64)), _full((1, 64)),
        _full((6, 64, 64)), _full((6, 1, 64)),
        _full((6, 64, 128)), _full((6, 1, 128)),
        _full((6, 128, 64)), _full((6, 1, 64)),
        _full((64, 256)), _full((1, 256)),
        _full((1, 256)),
        _full((256, 256)), _full((1, 256)),
        _full((256, 256)), _full((1, 256)),
        _full((256, 256)), _full((1, 256)),
        _full((256, 4)), _full((4, 256)),
        _full((256, 256)), _full((1, 256)),
        _full((256, 64)), _full((1, 64)),
        _full((1, 64)), _full((1, 64)),
        _full((64, 1024)), _full((1, 1024)),
    ]
    return pl.pallas_call(
        _tc_body,
        grid=(G,),
        in_specs=in_specs,
        out_specs=pl.BlockSpec((1, 1024), lambda i: (0, 0)),
        out_shape=jax.ShapeDtypeStruct((1, 1024), jnp.float32),
        scratch_shapes=[
            pltpu.VMEM((1, 4), jnp.float32),
            pltpu.VMEM((1, 256), jnp.float32),
            pltpu.VMEM((1, 256), jnp.float32),
        ] + [
            pltpu.VMEM((64, 4), jnp.float32),
            pltpu.VMEM((1, 4), jnp.float32),
            pltpu.VMEM((64, 256), jnp.float32),
            pltpu.VMEM((1, 256), jnp.float32),
        ],
        compiler_params=pltpu.CompilerParams(
            dimension_semantics=("arbitrary",)),
        interpret=interpret,
    )


def kernel(x, params, edge_index):
    row = edge_index[0]
    pad_idx = N + (jnp.arange(E_PAD - E, dtype=jnp.int32) % (NP - N))
    row_pad = jnp.concatenate([row, pad_idx])
    hist = _sc_hist_build()(row_pad)
    cnt0 = hist[:NP].reshape(NP, 1)
    cnt1 = hist[NP:].reshape(NP, 1)

    x_pad = jnp.pad(x, ((0, NP - N), (0, 0)))
    p = params
    lys = p['layers']
    w1s = jnp.stack([l['w1'] for l in lys])
    b1s = jnp.stack([l['b1'].reshape(1, -1) for l in lys])
    w2s = jnp.stack([l['w2'] for l in lys])
    b2s = jnp.stack([l['b2'].reshape(1, -1) for l in lys])
    w3s = jnp.stack([l['w3'] for l in lys])
    b3s = jnp.stack([l['b3'].reshape(1, -1) for l in lys])

    heads = jnp.arange(256, dtype=jnp.int32) // 64
    head_pool = (heads[:, None] == jnp.arange(4)[None, :]).astype(jnp.float32)
    head_bcast = head_pool.T
    msk = jnp.where(jnp.arange(NP) < N, 0.0, -1e30).astype(jnp.float32).reshape(NP, 1)

    return _tc_build()(
        x_pad, cnt0, cnt1, msk,
        p['embed_w'], p['embed_b'].reshape(1, -1),
        w1s, b1s, w2s, b2s, w3s, b3s,
        p['pool_w'], p['pool_b'].reshape(1, -1),
        p['query'],
        p['wq'], p['bq'].reshape(1, -1),
        p['wk'], p['bk'].reshape(1, -1),
        p['wv'], p['bv'].reshape(1, -1),
        head_pool, head_bcast,
        p['wo'], p['bo'].reshape(1, -1),
        p['pw1'], p['pb1'].reshape(1, -1),
        p['ln_g'].reshape(1, -1), p['ln_b'].reshape(1, -1),
        p['pw2'], p['pb2'].reshape(1, -1),
    )
